# baseline (speedup 1.0000x reference)
import functools

import jax
import jax.numpy as jnp
from jax import lax
from jax.experimental import pallas as pl
from jax.experimental.pallas import tpu as pltpu
from jax.experimental.pallas import tpu_sc as plsc

N_PROT = 10000
N_PATH = 10000
N_EDGE = 320000
D = 128

NS = 16
CHUNK = 64
NBUF = 4
NPH = 8
PH = 40
NG = PH // NBUF
NCHT = NS * NPH * PH
E_PAD = NCHT * CHUNK
R_TILE = 640
R_PAD = NS * R_TILE


def _mm_body(h_ref, w_ref, o_ref):
    o_ref[...] = jnp.dot(h_ref[...], w_ref[...],
                         preferred_element_type=jnp.float32)


def _fin_body(p_ref, b_ref, o_ref):
    o_ref[...] = jnp.maximum(p_ref[...] + b_ref[...], 0.0)


def _sc_body(msg_hbm, src_hbm, dst_hbm, out_hbm,
             src_v, dst_v, bufs, sem_g, sem_s, agg):
    s = lax.axis_index("s")

    zv = jnp.zeros((16,), jnp.float32)
    b0 = bufs[0]

    def zrow(r, _):
        for k8 in range(D // 16):
            b0[r, pl.ds(k8 * 16, 16)] = zv
        return 0

    lax.fori_loop(0, CHUNK, zrow, 0)
    for j in range(R_TILE // CHUNK):
        pltpu.sync_copy(b0, agg.at[pl.ds(s * R_TILE + j * CHUNK, CHUNK)])
    plsc.subcore_barrier()

    def gather(j, b):
        pltpu.async_copy(msg_hbm.at[src_v.at[j]], bufs[b], sem_g[b])

    def wait_gather(b):
        pltpu.make_async_copy(msg_hbm.at[src_v.at[0]], bufs[b],
                              sem_g[b]).wait()

    def scatter(j, b):
        pltpu.async_copy(bufs[b], agg.at[dst_v.at[j]], sem_s[b], add=True)

    def wait_scatter(b):
        pltpu.make_async_copy(bufs[b], agg.at[dst_v.at[0]], sem_s[b]).wait()

    for p in range(NPH):
        if p > 0:
            for b in range(NBUF):
                wait_scatter(b)
        row0 = (s * NPH + p) * PH
        pltpu.sync_copy(src_hbm.at[pl.ds(row0, PH)], src_v)
        pltpu.sync_copy(dst_hbm.at[pl.ds(row0, PH)], dst_v)

        for b in range(NBUF):
            gather(b, b)

        def body(i, _):
            j = NBUF * i
            for b in range(NBUF):
                wait_gather(b)
                scatter(j + b, b)
            j4 = j + NBUF
            for b in range(NBUF):
                wait_scatter(b)
                gather(j4 + b, b)
            return 0

        lax.fori_loop(0, NG - 1, body, 0)

        for b in range(NBUF):
            wait_gather(b)
            scatter((NG - 1) * NBUF + b, b)

    for b in range(NBUF):
        wait_scatter(b)
    plsc.subcore_barrier()
    pltpu.sync_copy(agg.at[pl.ds(s * R_TILE, R_TILE)],
                    out_hbm.at[pl.ds(s * R_TILE, R_TILE)])


_sc_scatter = functools.partial(
    pl.kernel,
    out_type=jax.ShapeDtypeStruct((R_PAD, D), jnp.float32),
    mesh=plsc.VectorSubcoreMesh(core_axis_name="c", subcore_axis_name="s",
                                num_cores=1, num_subcores=NS),
    scratch_types=[
        pltpu.VMEM((PH, CHUNK), jnp.int32),
        pltpu.VMEM((PH, CHUNK), jnp.int32),
        [pltpu.VMEM((CHUNK, D), jnp.float32) for _ in range(NBUF)],
        [pltpu.SemaphoreType.DMA for _ in range(NBUF)],
        [pltpu.SemaphoreType.DMA for _ in range(NBUF)],
        pltpu.VMEM_SHARED((R_PAD, D), jnp.float32),
    ],
)(_sc_body)


def kernel(h_p, h_path, edge_index, W, b):
    src = edge_index[0].astype(jnp.int32)
    dst = edge_index[1].astype(jnp.int32)
    pad = E_PAD - N_EDGE
    src = jnp.concatenate([src, jnp.zeros((pad,), jnp.int32)])
    pad_dst = N_PROT + (jnp.arange(pad, dtype=jnp.int32) % (R_PAD - N_PROT))
    dst = jnp.concatenate([dst, pad_dst])
    src3 = src.reshape(NCHT, CHUNK)
    dst3 = dst.reshape(NCHT, CHUNK)

    msg = pl.pallas_call(
        _mm_body,
        grid=(10,),
        in_specs=[pl.BlockSpec((N_PATH // 10, D), lambda i: (i, 0)),
                  pl.BlockSpec((D, D), lambda i: (0, 0))],
        out_specs=pl.BlockSpec((N_PATH // 10, D), lambda i: (i, 0)),
        out_shape=jax.ShapeDtypeStruct((N_PATH, D), jnp.float32),
    )(h_path, W)

    agg = _sc_scatter(msg, src3, dst3)

    out = pl.pallas_call(
        _fin_body,
        grid=(10,),
        in_specs=[pl.BlockSpec((N_PROT // 10, D), lambda i: (i, 0)),
                  pl.BlockSpec((1, D), lambda i: (0, 0))],
        out_specs=pl.BlockSpec((N_PROT // 10, D), lambda i: (i, 0)),
        out_shape=jax.ShapeDtypeStruct((N_PROT, D), jnp.float32),
    )(agg, b.reshape(1, D))
    return out


# speedup vs baseline: 1.0221x; 1.0221x over previous
import functools

import jax
import jax.numpy as jnp
from jax import lax
from jax.experimental import pallas as pl
from jax.experimental.pallas import tpu as pltpu
from jax.experimental.pallas import tpu_sc as plsc

N_PROT = 10000
N_PATH = 10000
N_EDGE = 320000
D = 128

NC = 2
NS = 16
CHUNK = 128
NPH = 4
PH = 40
NCHT = NS * NPH * PH
E_PAD = NCHT * CHUNK
R_TILE = 640
R_PAD = NS * R_TILE


def _mm_body(h_ref, w_ref, o_ref):
    o_ref[...] = jnp.dot(h_ref[...], w_ref[...],
                         preferred_element_type=jnp.float32)


def _fin_body(p_ref, b_ref, o_ref):
    o_ref[...] = jnp.maximum(p_ref[...] + b_ref[...], 0.0)


def _sc_body(msg_hbm, src_hbm, dst_hbm, out_hbm,
             src_v, dst_v, buf0, buf1, agg, sem0, sem1):
    c = lax.axis_index("c")
    s = lax.axis_index("s")

    @pl.when(c == 0)
    def _work():
        zv = jnp.zeros((16,), jnp.float32)

        def zrow(r, _):
            for k8 in range(D // 16):
                buf0[r, pl.ds(k8 * 16, 16)] = zv
            return 0

        lax.fori_loop(0, CHUNK, zrow, 0)
        for j in range(R_TILE // CHUNK):
            pltpu.sync_copy(buf0, agg.at[pl.ds(s * R_TILE + j * CHUNK, CHUNK)])
        plsc.subcore_barrier()

        for p in range(NPH):
            row0 = (s * NPH + p) * PH
            pltpu.sync_copy(src_hbm.at[pl.ds(row0, PH)], src_v)
            pltpu.sync_copy(dst_hbm.at[pl.ds(row0, PH)], dst_v)

            pltpu.async_copy(msg_hbm.at[src_v.at[0]], buf0, sem0)
            pltpu.async_copy(msg_hbm.at[src_v.at[1]], buf1, sem1)

            def body(k, _):
                j = 2 * k
                pltpu.make_async_copy(msg_hbm.at[src_v.at[j]], buf0,
                                      sem0).wait()
                pltpu.sync_copy(buf0, agg.at[dst_v.at[j]], add=True)
                pltpu.async_copy(msg_hbm.at[src_v.at[j + 2]], buf0, sem0)
                pltpu.make_async_copy(msg_hbm.at[src_v.at[j + 1]], buf1,
                                      sem1).wait()
                pltpu.sync_copy(buf1, agg.at[dst_v.at[j + 1]], add=True)
                pltpu.async_copy(msg_hbm.at[src_v.at[j + 3]], buf1, sem1)
                return 0

            lax.fori_loop(0, PH // 2 - 1, body, 0)

            pltpu.make_async_copy(msg_hbm.at[src_v.at[PH - 2]], buf0,
                                  sem0).wait()
            pltpu.sync_copy(buf0, agg.at[dst_v.at[PH - 2]], add=True)
            pltpu.make_async_copy(msg_hbm.at[src_v.at[PH - 1]], buf1,
                                  sem1).wait()
            pltpu.sync_copy(buf1, agg.at[dst_v.at[PH - 1]], add=True)

        plsc.subcore_barrier()
        pltpu.sync_copy(agg.at[pl.ds(s * R_TILE, R_TILE)],
                        out_hbm.at[pl.ds(s * R_TILE, R_TILE)])


_sc_scatter = functools.partial(
    pl.kernel,
    out_type=jax.ShapeDtypeStruct((R_PAD, D), jnp.float32),
    mesh=plsc.VectorSubcoreMesh(core_axis_name="c", subcore_axis_name="s",
                                num_cores=NC, num_subcores=NS),
    scratch_types=[
        pltpu.VMEM((PH, CHUNK), jnp.int32),
        pltpu.VMEM((PH, CHUNK), jnp.int32),
        pltpu.VMEM((CHUNK, D), jnp.float32),
        pltpu.VMEM((CHUNK, D), jnp.float32),
        pltpu.VMEM_SHARED((R_PAD, D), jnp.float32),
        pltpu.SemaphoreType.DMA,
        pltpu.SemaphoreType.DMA,
    ],
)(_sc_body)


def kernel(h_p, h_path, edge_index, W, b):
    src = edge_index[0].astype(jnp.int32)
    dst = edge_index[1].astype(jnp.int32)
    pad = E_PAD - N_EDGE
    src = jnp.concatenate([src, jnp.zeros((pad,), jnp.int32)])
    pad_dst = N_PROT + (jnp.arange(pad, dtype=jnp.int32) % (R_PAD - N_PROT))
    dst = jnp.concatenate([dst, pad_dst])
    src3 = src.reshape(NCHT, CHUNK)
    dst3 = dst.reshape(NCHT, CHUNK)

    msg = pl.pallas_call(
        _mm_body,
        grid=(10,),
        in_specs=[pl.BlockSpec((N_PATH // 10, D), lambda i: (i, 0)),
                  pl.BlockSpec((D, D), lambda i: (0, 0))],
        out_specs=pl.BlockSpec((N_PATH // 10, D), lambda i: (i, 0)),
        out_shape=jax.ShapeDtypeStruct((N_PATH, D), jnp.float32),
    )(h_path, W)

    agg = _sc_scatter(msg, src3, dst3)

    out = pl.pallas_call(
        _fin_body,
        grid=(10,),
        in_specs=[pl.BlockSpec((N_PROT // 10, D), lambda i: (i, 0)),
                  pl.BlockSpec((1, D), lambda i: (0, 0))],
        out_specs=pl.BlockSpec((N_PROT // 10, D), lambda i: (i, 0)),
        out_shape=jax.ShapeDtypeStruct((N_PROT, D), jnp.float32),
    )(agg, b.reshape(1, D))
    return out


# speedup vs baseline: 1.0500x; 1.0273x over previous
import functools

import jax
import jax.numpy as jnp
from jax import lax
from jax.experimental import pallas as pl
from jax.experimental.pallas import tpu as pltpu
from jax.experimental.pallas import tpu_sc as plsc

N_PROT = 10000
N_PATH = 10000
N_EDGE = 320000
D = 128

NC = 2
NS = 16
CHUNK = 128
NPH = 4
PH = 40
NCHT = NS * NPH * PH
E_PAD = NCHT * CHUNK
R_TILE = 640
R_PAD = NS * R_TILE


def _mm_body(h_ref, w_ref, o_ref):
    o_ref[...] = jnp.dot(h_ref[...], w_ref[...],
                         preferred_element_type=jnp.float32)


def _fin_body(p_ref, b_ref, o_ref):
    o_ref[...] = jnp.maximum(p_ref[...] + b_ref[...], 0.0)


def _sc_body(msg_hbm, src_hbm, dst_hbm, out_hbm,
             src_v, dst_v, buf0, buf1, agg, sem0, sem1):
    c = lax.axis_index("c")
    s = lax.axis_index("s")

    @pl.when(c == 0)
    def _work():
        zv = jnp.zeros((16,), jnp.float32)

        def zrow(r, _):
            for k8 in range(D // 16):
                buf0[r, pl.ds(k8 * 16, 16)] = zv
            return 0

        lax.fori_loop(0, CHUNK, zrow, 0)
        for j in range(R_TILE // CHUNK):
            pltpu.sync_copy(buf0, agg.at[pl.ds(s * R_TILE + j * CHUNK, CHUNK)])
        plsc.subcore_barrier()

        def phase(p, _):
            row0 = (s * NPH + p) * PH
            pltpu.sync_copy(src_hbm.at[pl.ds(row0, PH)], src_v)
            pltpu.sync_copy(dst_hbm.at[pl.ds(row0, PH)], dst_v)

            pltpu.async_copy(msg_hbm.at[src_v.at[0]], buf0, sem0)
            pltpu.async_copy(msg_hbm.at[src_v.at[1]], buf1, sem1)

            def body(k, _):
                j = 2 * k
                pltpu.make_async_copy(msg_hbm.at[src_v.at[j]], buf0,
                                      sem0).wait()
                pltpu.sync_copy(buf0, agg.at[dst_v.at[j]], add=True)
                pltpu.async_copy(msg_hbm.at[src_v.at[j + 2]], buf0, sem0)
                pltpu.make_async_copy(msg_hbm.at[src_v.at[j + 1]], buf1,
                                      sem1).wait()
                pltpu.sync_copy(buf1, agg.at[dst_v.at[j + 1]], add=True)
                pltpu.async_copy(msg_hbm.at[src_v.at[j + 3]], buf1, sem1)
                return 0

            lax.fori_loop(0, PH // 2 - 1, body, 0)

            pltpu.make_async_copy(msg_hbm.at[src_v.at[PH - 2]], buf0,
                                  sem0).wait()
            pltpu.sync_copy(buf0, agg.at[dst_v.at[PH - 2]], add=True)
            pltpu.make_async_copy(msg_hbm.at[src_v.at[PH - 1]], buf1,
                                  sem1).wait()
            pltpu.sync_copy(buf1, agg.at[dst_v.at[PH - 1]], add=True)
            return 0

        lax.fori_loop(0, NPH, phase, 0)

        plsc.subcore_barrier()
        pltpu.sync_copy(agg.at[pl.ds(s * R_TILE, R_TILE)],
                        out_hbm.at[pl.ds(s * R_TILE, R_TILE)])


_sc_scatter = functools.partial(
    pl.kernel,
    out_type=jax.ShapeDtypeStruct((R_PAD, D), jnp.float32),
    mesh=plsc.VectorSubcoreMesh(core_axis_name="c", subcore_axis_name="s",
                                num_cores=NC, num_subcores=NS),
    scratch_types=[
        pltpu.VMEM((PH, CHUNK), jnp.int32),
        pltpu.VMEM((PH, CHUNK), jnp.int32),
        pltpu.VMEM((CHUNK, D), jnp.float32),
        pltpu.VMEM((CHUNK, D), jnp.float32),
        pltpu.VMEM_SHARED((R_PAD, D), jnp.float32),
        pltpu.SemaphoreType.DMA,
        pltpu.SemaphoreType.DMA,
    ],
)(_sc_body)


def kernel(h_p, h_path, edge_index, W, b):
    src = edge_index[0].astype(jnp.int32)
    dst = edge_index[1].astype(jnp.int32)
    pad = E_PAD - N_EDGE
    src = jnp.concatenate([src, jnp.zeros((pad,), jnp.int32)])
    pad_dst = N_PROT + (jnp.arange(pad, dtype=jnp.int32) % (R_PAD - N_PROT))
    dst = jnp.concatenate([dst, pad_dst])
    src3 = src.reshape(NCHT, CHUNK)
    dst3 = dst.reshape(NCHT, CHUNK)

    msg = pl.pallas_call(
        _mm_body,
        grid=(10,),
        in_specs=[pl.BlockSpec((N_PATH // 10, D), lambda i: (i, 0)),
                  pl.BlockSpec((D, D), lambda i: (0, 0))],
        out_specs=pl.BlockSpec((N_PATH // 10, D), lambda i: (i, 0)),
        out_shape=jax.ShapeDtypeStruct((N_PATH, D), jnp.float32),
    )(h_path, W)

    agg = _sc_scatter(msg, src3, dst3)

    out = pl.pallas_call(
        _fin_body,
        grid=(10,),
        in_specs=[pl.BlockSpec((N_PROT // 10, D), lambda i: (i, 0)),
                  pl.BlockSpec((1, D), lambda i: (0, 0))],
        out_specs=pl.BlockSpec((N_PROT // 10, D), lambda i: (i, 0)),
        out_shape=jax.ShapeDtypeStruct((N_PROT, D), jnp.float32),
    )(agg, b.reshape(1, D))
    return out


# speedup vs baseline: 1.2621x; 1.2020x over previous
import functools

import jax
import jax.numpy as jnp
import numpy as np
from jax import lax
from jax.experimental import pallas as pl
from jax.experimental.pallas import tpu as pltpu
from jax.experimental.pallas import tpu_sc as plsc

N_PROT = 10000
N_PATH = 10000
N_EDGE = 320000
D = 128

NC = 2
NS = 16
CHUNK = 128
NPH = 4
PH = 40
NCHT = NS * NPH * PH
E_PAD = NCHT * CHUNK
R_TILE = 656
R_PAD = NS * R_TILE


def _mm_body(h_ref, w_ref, o_ref):
    o_ref[...] = jnp.dot(h_ref[...], w_ref[...],
                         preferred_element_type=jnp.float32)


def _fin_body(p_ref, b_ref, o_ref):
    o_ref[...] = jnp.maximum(p_ref[...] + b_ref[...], 0.0)


def _sc_body(msg_hbm, src_hbm, dst_hbm, out_hbm,
             src_v, dst_v, buf0, buf1, agg, sem0, sem1):
    c = lax.axis_index("c")
    s = lax.axis_index("s")

    @pl.when(c == 0)
    def _work():
        zv = jnp.zeros((16,), jnp.float32)

        def zrow(r, _):
            for k8 in range(D // 16):
                buf0[r, pl.ds(k8 * 16, 16)] = zv
            return 0

        lax.fori_loop(0, CHUNK, zrow, 0)
        for j in range(R_TILE // CHUNK):
            pltpu.sync_copy(buf0, agg.at[pl.ds(s * R_TILE + j * CHUNK, CHUNK)])
        rem = R_TILE % CHUNK
        if rem:
            pltpu.sync_copy(
                buf0.at[pl.ds(0, rem)],
                agg.at[pl.ds(s * R_TILE + (R_TILE // CHUNK) * CHUNK, rem)])
        plsc.subcore_barrier()

        def phase(p, _):
            row0 = (s * NPH + p) * PH
            pltpu.sync_copy(src_hbm.at[pl.ds(row0, PH)], src_v)
            pltpu.sync_copy(dst_hbm.at[pl.ds(row0, PH)], dst_v)

            pltpu.async_copy(msg_hbm.at[src_v.at[0]], buf0, sem0)
            pltpu.async_copy(msg_hbm.at[src_v.at[1]], buf1, sem1)

            def body(k, _):
                j = 2 * k
                pltpu.make_async_copy(msg_hbm.at[src_v.at[j]], buf0,
                                      sem0).wait()
                pltpu.sync_copy(buf0, agg.at[dst_v.at[j]], add=True)
                pltpu.async_copy(msg_hbm.at[src_v.at[j + 2]], buf0, sem0)
                pltpu.make_async_copy(msg_hbm.at[src_v.at[j + 1]], buf1,
                                      sem1).wait()
                pltpu.sync_copy(buf1, agg.at[dst_v.at[j + 1]], add=True)
                pltpu.async_copy(msg_hbm.at[src_v.at[j + 3]], buf1, sem1)
                return 0

            lax.fori_loop(0, PH // 2 - 1, body, 0)

            pltpu.make_async_copy(msg_hbm.at[src_v.at[PH - 2]], buf0,
                                  sem0).wait()
            pltpu.sync_copy(buf0, agg.at[dst_v.at[PH - 2]], add=True)
            pltpu.make_async_copy(msg_hbm.at[src_v.at[PH - 1]], buf1,
                                  sem1).wait()
            pltpu.sync_copy(buf1, agg.at[dst_v.at[PH - 1]], add=True)
            return 0

        lax.fori_loop(0, NPH, phase, 0)

        plsc.subcore_barrier()
        pltpu.sync_copy(agg.at[pl.ds(s * R_TILE, R_TILE)],
                        out_hbm.at[pl.ds(s * R_TILE, R_TILE)])


_sc_scatter = functools.partial(
    pl.kernel,
    out_type=jax.ShapeDtypeStruct((R_PAD, D), jnp.float32),
    mesh=plsc.VectorSubcoreMesh(core_axis_name="c", subcore_axis_name="s",
                                num_cores=NC, num_subcores=NS),
    scratch_types=[
        pltpu.VMEM((PH, CHUNK), jnp.int32),
        pltpu.VMEM((PH, CHUNK), jnp.int32),
        pltpu.VMEM((CHUNK, D), jnp.float32),
        pltpu.VMEM((CHUNK, D), jnp.float32),
        pltpu.VMEM_SHARED((R_PAD, D), jnp.float32),
        pltpu.SemaphoreType.DMA,
        pltpu.SemaphoreType.DMA,
    ],
)(_sc_body)


_PAD_N = E_PAD - N_EDGE
_pad_pos = np.round(np.linspace(0, E_PAD - 1, _PAD_N)).astype(np.int64)
_mask = np.zeros(E_PAD, dtype=bool)
_mask[_pad_pos] = True
_order = np.empty(E_PAD, dtype=np.int32)
_order[~_mask] = np.arange(N_EDGE, dtype=np.int32)
_order[_mask] = N_EDGE + np.arange(_PAD_N, dtype=np.int32)
_PAD_DST = N_PROT + (np.arange(_PAD_N, dtype=np.int32) % (R_PAD - N_PROT))


def kernel(h_p, h_path, edge_index, W, b):
    src = edge_index[0].astype(jnp.int32)
    dst = edge_index[1].astype(jnp.int32)
    order = jnp.asarray(_order)
    src = jnp.concatenate([src, jnp.zeros((_PAD_N,), jnp.int32)])[order]
    dst = jnp.concatenate([dst, jnp.asarray(_PAD_DST)])[order]
    src3 = src.reshape(NCHT, CHUNK)
    dst3 = dst.reshape(NCHT, CHUNK)

    msg = pl.pallas_call(
        _mm_body,
        grid=(10,),
        in_specs=[pl.BlockSpec((N_PATH // 10, D), lambda i: (i, 0)),
                  pl.BlockSpec((D, D), lambda i: (0, 0))],
        out_specs=pl.BlockSpec((N_PATH // 10, D), lambda i: (i, 0)),
        out_shape=jax.ShapeDtypeStruct((N_PATH, D), jnp.float32),
    )(h_path, W)

    agg = _sc_scatter(msg, src3, dst3)

    out = pl.pallas_call(
        _fin_body,
        grid=(10,),
        in_specs=[pl.BlockSpec((N_PROT // 10, D), lambda i: (i, 0)),
                  pl.BlockSpec((1, D), lambda i: (0, 0))],
        out_specs=pl.BlockSpec((N_PROT // 10, D), lambda i: (i, 0)),
        out_shape=jax.ShapeDtypeStruct((N_PROT, D), jnp.float32),
    )(agg, b.reshape(1, D))
    return out
